# baseline (speedup 1.0000x reference)
import jax
import jax.numpy as jnp
from jax.experimental import pallas as pl
from jax.experimental.pallas import tpu as pltpu

_N_ITER = 40


def _dot3(ah, al, ch, cl):
    return (jnp.dot(ah, ch, preferred_element_type=jnp.float32)
            + jnp.dot(ah, cl, preferred_element_type=jnp.float32)
            + jnp.dot(al, ch, preferred_element_type=jnp.float32))


def _split(v):
    hi = v.astype(jnp.bfloat16)
    lo = (v - hi.astype(jnp.float32)).astype(jnp.bfloat16)
    return hi, lo


def _prop_body(Ah_ref, Al_ref, xm_ref, out_ref, h0, l0, h1, l1, acc):
    t = pl.program_id(0)
    i = pl.program_id(1)
    k = pl.program_id(2)
    ni = pl.num_programs(1)
    nk = pl.num_programs(2)
    bi = acc.shape[0]
    bk = Ah_ref.shape[1]
    even = t % 2 == 0

    @pl.when((t == 0) & (i == 0) & (k == 0))
    def _init():
        hi, lo = _split(xm_ref[...])
        h0[...] = hi
        l0[...] = lo

    @pl.when(k == 0)
    def _zero_acc():
        acc[...] = jnp.zeros_like(acc)

    ks = pl.ds(k * bk, bk)

    @pl.when(even)
    def _mm_even():
        acc[...] += _dot3(Ah_ref[...], Al_ref[...], h0[ks, :], l0[ks, :])

    @pl.when(~even)
    def _mm_odd():
        acc[...] += _dot3(Ah_ref[...], Al_ref[...], h1[ks, :], l1[ks, :])

    @pl.when(k == nk - 1)
    def _finish_row_block():
        isl = pl.ds(i * bi, bi)
        val = xm_ref[isl, :] + acc[...]
        out_ref[isl, :] = val
        hi, lo = _split(val)

        @pl.when(even)
        def _w_odd():
            h1[isl, :] = hi
            l1[isl, :] = lo

        @pl.when(~even)
        def _w_even():
            h0[isl, :] = hi
            l0[isl, :] = lo


def _run_prop(Ah, Al, xm, bi, bk):
    npad, d = xm.shape
    grid = (_N_ITER, npad // bi, npad // bk)
    return pl.pallas_call(
        _prop_body,
        grid=grid,
        in_specs=[
            pl.BlockSpec((bi, bk), lambda t, i, k: (i, k)),
            pl.BlockSpec((bi, bk), lambda t, i, k: (i, k)),
            pl.BlockSpec((npad, d), lambda t, i, k: (0, 0)),
        ],
        out_specs=pl.BlockSpec((npad, d), lambda t, i, k: (0, 0)),
        out_shape=jax.ShapeDtypeStruct((npad, d), jnp.float32),
        scratch_shapes=[
            pltpu.VMEM((npad, d), jnp.bfloat16),
            pltpu.VMEM((npad, d), jnp.bfloat16),
            pltpu.VMEM((npad, d), jnp.bfloat16),
            pltpu.VMEM((npad, d), jnp.bfloat16),
            pltpu.VMEM((bi, d), jnp.float32),
        ],
    )(Ah, Al, xm)


def kernel(x, edge_index, mask):
    n, d = x.shape
    row = edge_index[0].astype(jnp.int32)
    col = edge_index[1].astype(jnp.int32)
    npad = ((n + 1023) // 1024) * 1024

    ones = jnp.ones(row.shape, jnp.float32)
    deg = jnp.zeros((n,), jnp.float32).at[col].add(ones)
    dis = jnp.where(deg > 0, jax.lax.rsqrt(deg), 0.0)
    w = dis[row] * dis[col] * (1.0 - mask[row].astype(jnp.float32))

    A = jnp.zeros((npad, npad), jnp.float32).at[row, col].add(w)
    Ah = A.astype(jnp.bfloat16)
    Al = (A - Ah.astype(jnp.float32)).astype(jnp.bfloat16)
    xm = jnp.where(mask[:, None], x, 0.0).astype(jnp.float32)
    xm_p = jnp.zeros((npad, d), jnp.float32).at[:n].set(xm)

    out = _run_prop(Ah, Al, xm_p, 1024, 1024)
    return out[:n]


# speedup vs baseline: 1.2616x; 1.2616x over previous
import functools

import jax
import jax.numpy as jnp
from jax import lax
from jax.experimental import pallas as pl
from jax.experimental.pallas import tpu as pltpu
from jax.experimental.pallas import tpu_sc as plsc

_N_ITER = 40
_NC = 2
_NS = 16
_RPT = 320
_HALF = _NS * _RPT
_NP = _NC * _HALF
_CH = 128
_D = 128


def _sc_step_body(v_hbm, cols_hbm, rloc_hbm, cnts_hbm, a_hbm, b_hbm, zb_hbm,
                  vout_hbm, acc_sh, gbuf, av, bv, idxc, idxr, cntv, sem):
    c = lax.axis_index("c")
    s = lax.axis_index("s")
    wid = c * _NS + s
    base_l = s * _RPT
    base_g = wid * _RPT

    pltpu.sync_copy(zb_hbm, gbuf)
    for blk, bsz in ((0, 128), (128, 128), (256, 64)):
        pltpu.sync_copy(gbuf.at[pl.ds(0, bsz)],
                        acc_sh.at[pl.ds(base_l + blk, bsz)])
    plsc.subcore_barrier()

    pltpu.sync_copy(cnts_hbm, cntv)
    cnt_vec = cntv[...]
    nch = jnp.where(c == 0, cnt_vec[0], cnt_vec[1])
    my_n = jnp.maximum(0, (nch - s + _NS - 1) // _NS)

    def chunk_body(t, carry):
        ebase = (t * _NS + s) * _CH
        pltpu.sync_copy(cols_hbm.at[c, pl.ds(ebase, _CH)], idxc)
        pltpu.sync_copy(rloc_hbm.at[c, pl.ds(ebase, _CH)], idxr)
        pltpu.async_copy(
            v_hbm.at[plsc.Indices(idxc, ignored_value=-1)], gbuf, sem
        ).wait()
        pltpu.sync_copy(
            gbuf, acc_sh.at[plsc.Indices(idxr, ignored_value=-1)], add=True
        )
        return carry

    lax.fori_loop(0, my_n, chunk_body, 0)
    plsc.subcore_barrier()

    for blk, bsz in ((0, 128), (128, 128), (256, 64)):
        pltpu.sync_copy(acc_sh.at[pl.ds(base_l + blk, bsz)],
                        gbuf.at[pl.ds(0, bsz)])
        pltpu.sync_copy(a_hbm.at[pl.ds(base_g + blk, bsz)],
                        av.at[pl.ds(0, bsz)])
        pltpu.sync_copy(b_hbm.at[pl.ds(base_g + blk, bsz)],
                        bv.at[pl.ds(0, bsz)])

        def row_body(r, carry):
            for cc in range(_D // 16):
                sl = pl.ds(cc * 16, 16)
                av[r, sl] = av[r, sl] + bv[r, sl] * gbuf[r, sl]
            return carry

        lax.fori_loop(0, bsz, row_body, 0)
        pltpu.sync_copy(av.at[pl.ds(0, bsz)],
                        vout_hbm.at[pl.ds(base_g + blk, bsz)])


_sc_step = functools.partial(
    pl.kernel,
    out_type=jax.ShapeDtypeStruct((_NP, _D), jnp.float32),
    mesh=plsc.VectorSubcoreMesh(core_axis_name="c", subcore_axis_name="s"),
    scratch_types=[
        pltpu.VMEM_SHARED((_HALF, _D), jnp.float32),
        pltpu.VMEM((_CH, _D), jnp.float32),
        pltpu.VMEM((_CH, _D), jnp.float32),
        pltpu.VMEM((_CH, _D), jnp.float32),
        pltpu.VMEM((_CH,), jnp.int32),
        pltpu.VMEM((_CH,), jnp.int32),
        pltpu.VMEM((16,), jnp.int32),
        pltpu.SemaphoreType.DMA,
    ],
)(_sc_step_body)


def kernel(x, edge_index, mask):
    n, d = x.shape
    e = edge_index.shape[1]
    row = edge_index[0].astype(jnp.int32)
    col = edge_index[1].astype(jnp.int32)
    maskf = mask.astype(jnp.float32)

    ones = jnp.ones((e,), jnp.float32)
    deg = jnp.zeros((n,), jnp.float32).at[col].add(ones)
    dis = jnp.where(deg > 0, jax.lax.rsqrt(deg), 0.0)

    in0 = row < _HALF
    pos0 = jnp.cumsum(in0.astype(jnp.int32)) - 1
    n0 = pos0[-1] + 1
    pos1 = jnp.cumsum(1 - in0.astype(jnp.int32)) - 1
    pos = jnp.where(in0, pos0, e + pos1)
    cols2 = jnp.full((2 * e,), -1, jnp.int32).at[pos].set(col)
    rloc2 = jnp.full((2 * e,), -1, jnp.int32).at[pos].set(
        jnp.where(in0, row, row - _HALF))
    cols_p = cols2.reshape(2, e)
    rloc_p = rloc2.reshape(2, e)
    n1 = e - n0
    cnts = jnp.zeros((16,), jnp.int32).at[0].set(
        (n0 + _CH - 1) // _CH).at[1].set((n1 + _CH - 1) // _CH)

    xm = jnp.where(mask[:, None], x, 0.0).astype(jnp.float32)
    xm_p = jnp.zeros((_NP, d), jnp.float32).at[:n].set(xm)
    dis_p = jnp.zeros((_NP,), jnp.float32).at[:n].set(dis)
    nm_p = jnp.zeros((_NP,), jnp.float32).at[:n].set(1.0 - maskf)

    a_iter = xm_p * dis_p[:, None]
    b_iter = jnp.broadcast_to((nm_p * dis_p * dis_p)[:, None], (_NP, d))
    a_last = xm_p
    b_last = jnp.broadcast_to((nm_p * dis_p)[:, None], (_NP, d))
    zblk = jnp.zeros((_CH, _D), jnp.float32)

    v = lax.fori_loop(
        0, _N_ITER - 1,
        lambda _, vv: _sc_step(vv, cols_p, rloc_p, cnts, a_iter, b_iter, zblk),
        a_iter)
    out = _sc_step(v, cols_p, rloc_p, cnts, a_last, b_last, zblk)
    return out[:n]


# speedup vs baseline: 1.8986x; 1.5049x over previous
import functools

import jax
import jax.numpy as jnp
from jax import lax
from jax.experimental import pallas as pl
from jax.experimental.pallas import tpu as pltpu
from jax.experimental.pallas import tpu_sc as plsc

_N_ITER = 40
_NC = 2
_NS = 16
_RPT = 320
_HALF = _NS * _RPT
_NP = _NC * _HALF
_CH = 128
_D = 128


_W = 3


def _sc_step_body(v_hbm, cols_hbm, rloc_hbm, cnts_hbm, a_hbm, b_hbm, zb_hbm,
                  vout_hbm, acc_sh, gbuf, av, bv, idxcs, idxrs, cntv,
                  sem_i, sem_g, sem_s):
    c = lax.axis_index("c")
    s = lax.axis_index("s")
    wid = c * _NS + s
    base_l = s * _RPT
    base_g = wid * _RPT
    bufs = (gbuf, av, bv)

    pltpu.sync_copy(zb_hbm, gbuf)
    for blk, bsz in ((0, 128), (128, 128), (256, 64)):
        pltpu.sync_copy(gbuf.at[pl.ds(0, bsz)],
                        acc_sh.at[pl.ds(base_l + blk, bsz)])
    plsc.subcore_barrier()

    pltpu.sync_copy(cnts_hbm, cntv)
    cnt_vec = cntv[...]
    nch = jnp.where(c == 0, cnt_vec[0], cnt_vec[1])
    my_n = jnp.maximum(0, (nch - s + _NS - 1) // _NS)
    n_waves = (my_n + _W - 1) // _W

    def wave_body(w, carry):
        ts = [w * _W + b for b in range(_W)]
        valid = [t < my_n for t in ts]
        ebases = [(t * _NS + s) * _CH for t in ts]
        for b in range(_W):
            @pl.when(valid[b])
            def _issue_idx(b=b):
                pltpu.async_copy(cols_hbm.at[c, pl.ds(ebases[b], _CH)],
                                 idxcs.at[b], sem_i)
                pltpu.async_copy(rloc_hbm.at[c, pl.ds(ebases[b], _CH)],
                                 idxrs.at[b], sem_i)
        for b in range(_W):
            @pl.when(valid[b])
            def _wait_idx(b=b):
                pltpu.make_async_copy(cols_hbm.at[c, pl.ds(ebases[b], _CH)],
                                      idxcs.at[b], sem_i).wait()
                pltpu.make_async_copy(rloc_hbm.at[c, pl.ds(ebases[b], _CH)],
                                      idxrs.at[b], sem_i).wait()
        for b in range(_W):
            @pl.when(valid[b])
            def _issue_gather(b=b):
                pltpu.async_copy(
                    v_hbm.at[plsc.Indices(idxcs.at[b], ignored_value=-1)],
                    bufs[b], sem_g)
        for b in range(_W):
            @pl.when(valid[b])
            def _gather_scatter(b=b):
                pltpu.make_async_copy(
                    v_hbm.at[plsc.Indices(idxcs.at[b], ignored_value=-1)],
                    bufs[b], sem_g).wait()
                pltpu.async_copy(
                    bufs[b],
                    acc_sh.at[plsc.Indices(idxrs.at[b], ignored_value=-1)],
                    sem_s, add=True)
        for b in range(_W):
            @pl.when(valid[b])
            def _wait_scatter(b=b):
                pltpu.make_async_copy(
                    bufs[b],
                    acc_sh.at[plsc.Indices(idxrs.at[b], ignored_value=-1)],
                    sem_s).wait()
        return carry

    lax.fori_loop(0, n_waves, wave_body, 0)
    plsc.subcore_barrier()

    for blk, bsz in ((0, 128), (128, 128), (256, 64)):
        pltpu.sync_copy(acc_sh.at[pl.ds(base_l + blk, bsz)],
                        gbuf.at[pl.ds(0, bsz)])
        pltpu.sync_copy(a_hbm.at[pl.ds(base_g + blk, bsz)],
                        av.at[pl.ds(0, bsz)])
        pltpu.sync_copy(b_hbm.at[pl.ds(base_g + blk, bsz)],
                        bv.at[pl.ds(0, bsz)])

        def row_body(r, carry):
            for cc in range(_D // 16):
                sl = pl.ds(cc * 16, 16)
                av[r, sl] = av[r, sl] + bv[r, sl] * gbuf[r, sl]
            return carry

        lax.fori_loop(0, bsz, row_body, 0)
        pltpu.sync_copy(av.at[pl.ds(0, bsz)],
                        vout_hbm.at[pl.ds(base_g + blk, bsz)])


_sc_step = functools.partial(
    pl.kernel,
    out_type=jax.ShapeDtypeStruct((_NP, _D), jnp.float32),
    mesh=plsc.VectorSubcoreMesh(core_axis_name="c", subcore_axis_name="s"),
    scratch_types=[
        pltpu.VMEM_SHARED((_HALF, _D), jnp.float32),
        pltpu.VMEM((_CH, _D), jnp.float32),
        pltpu.VMEM((_CH, _D), jnp.float32),
        pltpu.VMEM((_CH, _D), jnp.float32),
        pltpu.VMEM((_W, _CH), jnp.int32),
        pltpu.VMEM((_W, _CH), jnp.int32),
        pltpu.VMEM((16,), jnp.int32),
        pltpu.SemaphoreType.DMA,
        pltpu.SemaphoreType.DMA,
        pltpu.SemaphoreType.DMA,
    ],
)(_sc_step_body)


def kernel(x, edge_index, mask):
    n, d = x.shape
    e = edge_index.shape[1]
    row = edge_index[0].astype(jnp.int32)
    col = edge_index[1].astype(jnp.int32)
    maskf = mask.astype(jnp.float32)

    ones = jnp.ones((e,), jnp.float32)
    deg = jnp.zeros((n,), jnp.float32).at[col].add(ones)
    dis = jnp.where(deg > 0, jax.lax.rsqrt(deg), 0.0)

    in0 = row < _HALF
    pos0 = jnp.cumsum(in0.astype(jnp.int32)) - 1
    n0 = pos0[-1] + 1
    pos1 = jnp.cumsum(1 - in0.astype(jnp.int32)) - 1
    pos = jnp.where(in0, pos0, e + pos1)
    cols2 = jnp.full((2 * e,), -1, jnp.int32).at[pos].set(col)
    rloc2 = jnp.full((2 * e,), -1, jnp.int32).at[pos].set(
        jnp.where(in0, row, row - _HALF))
    cols_p = cols2.reshape(2, e)
    rloc_p = rloc2.reshape(2, e)
    n1 = e - n0
    cnts = jnp.zeros((16,), jnp.int32).at[0].set(
        (n0 + _CH - 1) // _CH).at[1].set((n1 + _CH - 1) // _CH)

    xm = jnp.where(mask[:, None], x, 0.0).astype(jnp.float32)
    xm_p = jnp.zeros((_NP, d), jnp.float32).at[:n].set(xm)
    dis_p = jnp.zeros((_NP,), jnp.float32).at[:n].set(dis)
    nm_p = jnp.zeros((_NP,), jnp.float32).at[:n].set(1.0 - maskf)

    a_iter = xm_p * dis_p[:, None]
    b_iter = jnp.broadcast_to((nm_p * dis_p * dis_p)[:, None], (_NP, d))
    a_last = xm_p
    b_last = jnp.broadcast_to((nm_p * dis_p)[:, None], (_NP, d))
    zblk = jnp.zeros((_CH, _D), jnp.float32)

    v = lax.fori_loop(
        0, _N_ITER - 1,
        lambda _, vv: _sc_step(vv, cols_p, rloc_p, cnts, a_iter, b_iter, zblk),
        a_iter)
    out = _sc_step(v, cols_p, rloc_p, cnts, a_last, b_last, zblk)
    return out[:n]


# speedup vs baseline: 1.9675x; 1.0363x over previous
import functools

import jax
import jax.numpy as jnp
from jax import lax
from jax.experimental import pallas as pl
from jax.experimental.pallas import tpu as pltpu
from jax.experimental.pallas import tpu_sc as plsc

_N_ITER = 40
_NC = 2
_NS = 16
_RPT = 320
_HALF = _NS * _RPT
_NP = _NC * _HALF
_CH = 128
_D = 128


_W = 5


def _sc_step_body(v_hbm, cols_hbm, rloc_hbm, cnts_hbm, a_hbm, b_hbm, zb_hbm,
                  vout_hbm, acc_sh, gbuf, av, bv, g3, g4, idxcs, idxrs, cntv,
                  sem_i, sem_g, sem_s):
    c = lax.axis_index("c")
    s = lax.axis_index("s")
    wid = c * _NS + s
    base_l = s * _RPT
    base_g = wid * _RPT
    bufs = (gbuf, av, bv, g3, g4)

    pltpu.sync_copy(zb_hbm, gbuf)
    for blk, bsz in ((0, 128), (128, 128), (256, 64)):
        pltpu.sync_copy(gbuf.at[pl.ds(0, bsz)],
                        acc_sh.at[pl.ds(base_l + blk, bsz)])
    plsc.subcore_barrier()

    pltpu.sync_copy(cnts_hbm, cntv)
    cnt_vec = cntv[...]
    nch = jnp.where(c == 0, cnt_vec[0], cnt_vec[1])
    my_n = jnp.maximum(0, (nch - s + _NS - 1) // _NS)
    n_waves = (my_n + _W - 1) // _W

    def wave_body(w, carry):
        ts = [w * _W + b for b in range(_W)]
        valid = [t < my_n for t in ts]
        ebases = [(t * _NS + s) * _CH for t in ts]
        for b in range(_W):
            @pl.when(valid[b])
            def _issue_idx(b=b):
                pltpu.async_copy(cols_hbm.at[c, pl.ds(ebases[b], _CH)],
                                 idxcs.at[b], sem_i)
                pltpu.async_copy(rloc_hbm.at[c, pl.ds(ebases[b], _CH)],
                                 idxrs.at[b], sem_i)
        for b in range(_W):
            @pl.when(valid[b])
            def _wait_idx(b=b):
                pltpu.make_async_copy(cols_hbm.at[c, pl.ds(ebases[b], _CH)],
                                      idxcs.at[b], sem_i).wait()
                pltpu.make_async_copy(rloc_hbm.at[c, pl.ds(ebases[b], _CH)],
                                      idxrs.at[b], sem_i).wait()
        for b in range(_W):
            @pl.when(valid[b])
            def _issue_gather(b=b):
                pltpu.async_copy(
                    v_hbm.at[plsc.Indices(idxcs.at[b], ignored_value=-1)],
                    bufs[b], sem_g)
        for b in range(_W):
            @pl.when(valid[b])
            def _gather_scatter(b=b):
                pltpu.make_async_copy(
                    v_hbm.at[plsc.Indices(idxcs.at[b], ignored_value=-1)],
                    bufs[b], sem_g).wait()
                pltpu.async_copy(
                    bufs[b],
                    acc_sh.at[plsc.Indices(idxrs.at[b], ignored_value=-1)],
                    sem_s, add=True)
        for b in range(_W):
            @pl.when(valid[b])
            def _wait_scatter(b=b):
                pltpu.make_async_copy(
                    bufs[b],
                    acc_sh.at[plsc.Indices(idxrs.at[b], ignored_value=-1)],
                    sem_s).wait()
        return carry

    lax.fori_loop(0, n_waves, wave_body, 0)
    plsc.subcore_barrier()

    for blk, bsz in ((0, 128), (128, 128), (256, 64)):
        pltpu.sync_copy(acc_sh.at[pl.ds(base_l + blk, bsz)],
                        gbuf.at[pl.ds(0, bsz)])
        pltpu.sync_copy(a_hbm.at[pl.ds(base_g + blk, bsz)],
                        av.at[pl.ds(0, bsz)])
        pltpu.sync_copy(b_hbm.at[pl.ds(base_g + blk, bsz)],
                        bv.at[pl.ds(0, bsz)])

        def row_body(r, carry):
            for cc in range(_D // 16):
                sl = pl.ds(cc * 16, 16)
                av[r, sl] = av[r, sl] + bv[r, sl] * gbuf[r, sl]
            return carry

        lax.fori_loop(0, bsz, row_body, 0)
        pltpu.sync_copy(av.at[pl.ds(0, bsz)],
                        vout_hbm.at[pl.ds(base_g + blk, bsz)])


_sc_step = functools.partial(
    pl.kernel,
    out_type=jax.ShapeDtypeStruct((_NP, _D), jnp.float32),
    mesh=plsc.VectorSubcoreMesh(core_axis_name="c", subcore_axis_name="s"),
    scratch_types=[
        pltpu.VMEM_SHARED((_HALF, _D), jnp.float32),
        pltpu.VMEM((_CH, _D), jnp.float32),
        pltpu.VMEM((_CH, _D), jnp.float32),
        pltpu.VMEM((_CH, _D), jnp.float32),
        pltpu.VMEM((_CH, _D), jnp.float32),
        pltpu.VMEM((_CH, _D), jnp.float32),
        pltpu.VMEM((_W, _CH), jnp.int32),
        pltpu.VMEM((_W, _CH), jnp.int32),
        pltpu.VMEM((16,), jnp.int32),
        pltpu.SemaphoreType.DMA,
        pltpu.SemaphoreType.DMA,
        pltpu.SemaphoreType.DMA,
    ],
)(_sc_step_body)


def kernel(x, edge_index, mask):
    n, d = x.shape
    e = edge_index.shape[1]
    row = edge_index[0].astype(jnp.int32)
    col = edge_index[1].astype(jnp.int32)
    maskf = mask.astype(jnp.float32)

    ones = jnp.ones((e,), jnp.float32)
    deg = jnp.zeros((n,), jnp.float32).at[col].add(ones)
    dis = jnp.where(deg > 0, jax.lax.rsqrt(deg), 0.0)

    in0 = row < _HALF
    pos0 = jnp.cumsum(in0.astype(jnp.int32)) - 1
    n0 = pos0[-1] + 1
    pos1 = jnp.cumsum(1 - in0.astype(jnp.int32)) - 1
    pos = jnp.where(in0, pos0, e + pos1)
    cols2 = jnp.full((2 * e,), -1, jnp.int32).at[pos].set(col)
    rloc2 = jnp.full((2 * e,), -1, jnp.int32).at[pos].set(
        jnp.where(in0, row, row - _HALF))
    cols_p = cols2.reshape(2, e)
    rloc_p = rloc2.reshape(2, e)
    n1 = e - n0
    cnts = jnp.zeros((16,), jnp.int32).at[0].set(
        (n0 + _CH - 1) // _CH).at[1].set((n1 + _CH - 1) // _CH)

    xm = jnp.where(mask[:, None], x, 0.0).astype(jnp.float32)
    xm_p = jnp.zeros((_NP, d), jnp.float32).at[:n].set(xm)
    dis_p = jnp.zeros((_NP,), jnp.float32).at[:n].set(dis)
    nm_p = jnp.zeros((_NP,), jnp.float32).at[:n].set(1.0 - maskf)

    a_iter = xm_p * dis_p[:, None]
    b_iter = jnp.broadcast_to((nm_p * dis_p * dis_p)[:, None], (_NP, d))
    a_last = xm_p
    b_last = jnp.broadcast_to((nm_p * dis_p)[:, None], (_NP, d))
    zblk = jnp.zeros((_CH, _D), jnp.float32)

    v = lax.fori_loop(
        0, _N_ITER - 1,
        lambda _, vv: _sc_step(vv, cols_p, rloc_p, cnts, a_iter, b_iter, zblk),
        a_iter)
    out = _sc_step(v, cols_p, rloc_p, cnts, a_last, b_last, zblk)
    return out[:n]
